# baseline (speedup 1.0000x reference)
import functools

import jax
import jax.numpy as jnp
from jax import lax
from jax.experimental import pallas as pl
from jax.experimental.pallas import tpu as pltpu
from jax.experimental.pallas import tpu_sc as plsc

B = 16384
D = 16
LANES = 128
SUB = 8
TILE_WORDS = SUB * LANES
N_CORES = 2
N_SUBCORES = 16
NW = N_CORES * N_SUBCORES
BPW = B // NW
NGROUP = BPW // D
CHUNK = 128
NCHUNK = BPW // CHUNK


def _ncols(n_rows):
    return -(-n_rows // LANES)


QB = 61


def _dump_table(src3, dst, n_rows, wid, buf, sem_in, sem_out):
    ncols = _ncols(n_rows)
    per = ncols // NW
    extra = ncols - per * NW
    base_c = wid * per + jnp.minimum(wid, extra)
    batches = [(h, off, min(QB, per - off))
               for h in range(D // SUB) for off in range(0, per, QB)]

    def fire_ins(b, slot):
        h, off, s = batches[b]
        ins = []
        for q in range(s):
            start = pl.multiple_of((base_c + off + q) * LANES, LANES)
            ins.append(pltpu.async_copy(
                src3.at[h, :, pl.ds(start, LANES)], buf.at[slot, q], sem_in))
        return ins

    outs = []
    ins_cur = fire_ins(0, 0)
    for b in range(len(batches)):
        slot = b % 2
        ins_next = None
        if b + 1 < len(batches):
            if b >= 1:
                outs[b - 1].wait()
            ins_next = fire_ins(b + 1, 1 - slot)
        for cp in ins_cur:
            cp.wait()
        h, off, s = batches[b]
        outs.append(pltpu.async_copy(
            buf.at[slot, pl.ds(0, s)],
            dst.at[pl.ds(h * ncols + base_c + off, s)], sem_out))
        ins_cur = ins_next
    for cp in outs[-2:]:
        cp.wait()

    @pl.when(wid < extra)
    def _():
        for h in range(D // SUB):
            c = base_c + per
            start = pl.multiple_of(c * LANES, LANES)
            pltpu.async_copy(
                src3.at[h, :, pl.ds(start, LANES)], buf.at[0, 0],
                sem_in).wait()
            pltpu.async_copy(
                buf.at[0, pl.ds(0, 1)],
                dst.at[pl.ds(h * ncols + c, 1)], sem_out).wait()


def _dump_body(tf_hbm, tf_dump, buf, sem_in, sem_out):
    cid = lax.axis_index("c")
    sid = lax.axis_index("s")
    wid = sid * N_CORES + cid
    n_task = tf_hbm.shape[1]
    tf3 = tf_hbm.reshape(D // SUB, SUB, n_task)
    _dump_table(tf3, tf_dump, n_task, wid, buf, sem_in, sem_out)


def _make_gather_body(n_task, n_worker):
    ct = [(k // SUB) * _ncols(n_task) * TILE_WORDS + (k % SUB) * LANES
          for k in range(D)]

    def body(task_hbm, worker_hbm, tf_dump, wf_rm, out_hbm,
             raw_t, raw_w, idx_t, dst_t, dst_w, out_v, sem_t, sem_w):
        wid = lax.axis_index("s") * N_CORES + lax.axis_index("c")
        base = wid * BPW
        pltpu.sync_copy(task_hbm.at[pl.ds(base, BPW)], raw_t)
        pltpu.sync_copy(worker_hbm.at[pl.ds(base, BPW)], raw_w)

        def expand(g, carry):
            rt = raw_t[pl.ds(g * D, D)]
            ft = ((rt >> 7) << 10) + (rt & (LANES - 1))
            for k in range(D):
                idx_t[pl.ds(k * BPW + g * D, D)] = ft + ct[k]
            return carry

        lax.fori_loop(0, NGROUP, expand, 0)

        copies = []
        for c in range(NCHUNK):
            copies.append(pltpu.async_copy(
                wf_rm.at[raw_w.at[pl.ds(c * CHUNK, CHUNK)]],
                dst_w.at[pl.ds(c * CHUNK, CHUNK)], sem_w))
        for k in range(D):
            for c in range(NCHUNK):
                off = k * BPW + c * CHUNK
                copies.append(pltpu.async_copy(
                    tf_dump.at[idx_t.at[pl.ds(off, CHUNK)]],
                    dst_t.at[pl.ds(off, CHUNK)], sem_t))
        for cp in copies:
            cp.wait()

        lane = lax.iota(jnp.int32, D)

        def dot(g, carry):
            rows = lane + g * D
            acc = jnp.zeros((D,), jnp.float32)
            for k in range(D):
                a = dst_t[pl.ds(k * BPW + g * D, D)]
                b = plsc.load_gather(dst_w, [rows, jnp.full((D,), k, jnp.int32)])
                acc = acc + a * b
            out_v[pl.ds(g * D, D)] = 1.0 / (1.0 + jnp.exp(-acc))
            return carry

        lax.fori_loop(0, NGROUP, dot, 0)

        pltpu.sync_copy(out_v, out_hbm.at[pl.ds(base, BPW)])

    return body


@jax.jit
def kernel(task, worker, task_factors, worker_factors):
    n_task, _ = task_factors.shape
    n_worker, _ = worker_factors.shape
    nwords_t = (D // SUB) * _ncols(n_task) * TILE_WORDS
    tf_t = task_factors.T

    mesh = plsc.VectorSubcoreMesh(core_axis_name="c", subcore_axis_name="s")

    dump = functools.partial(
        pl.kernel,
        out_type=jax.ShapeDtypeStruct(
            (nwords_t // TILE_WORDS, SUB, LANES), jnp.float32),
        mesh=mesh,
        scratch_types=[
            pltpu.VMEM((2, QB, SUB, LANES), jnp.float32),
            pltpu.SemaphoreType.DMA,
            pltpu.SemaphoreType.DMA,
        ],
        compiler_params=pltpu.CompilerParams(needs_layout_passes=False),
    )(_dump_body)
    tf_dump = dump(tf_t)
    tf_dump = tf_dump.reshape(nwords_t)

    gather = functools.partial(
        pl.kernel,
        out_type=jax.ShapeDtypeStruct((B,), jnp.float32),
        mesh=mesh,
        scratch_types=[
            pltpu.VMEM((BPW,), jnp.int32),
            pltpu.VMEM((BPW,), jnp.int32),
            pltpu.VMEM((D * BPW,), jnp.int32),
            pltpu.VMEM((D * BPW,), jnp.float32),
            pltpu.VMEM((BPW, D), jnp.float32),
            pltpu.VMEM((BPW,), jnp.float32),
            pltpu.SemaphoreType.DMA,
            pltpu.SemaphoreType.DMA,
        ],
        compiler_params=pltpu.CompilerParams(
            needs_layout_passes=False, use_tc_tiling_on_sc=False),
    )(_make_gather_body(n_task, n_worker))
    return gather(task, worker, tf_dump, worker_factors)


# speedup vs baseline: 1.1073x; 1.1073x over previous
import functools

import jax
import jax.numpy as jnp
from jax import lax
from jax.experimental import pallas as pl
from jax.experimental.pallas import tpu as pltpu
from jax.experimental.pallas import tpu_sc as plsc

B = 16384
D = 16
LANES = 128
SUB = 8
TILE_WORDS = SUB * LANES
N_CORES = 2
N_SUBCORES = 16
NW = N_CORES * N_SUBCORES
BPW = B // NW
NGROUP = BPW // D
CHUNK = 128
NCHUNK = BPW // CHUNK


def _ncols(n_rows):
    return -(-n_rows // LANES)


QB = 61


def _dump_table(src3, dst, n_rows, wid, buf, sem_in, sem_out):
    ncols = _ncols(n_rows)
    per = ncols // NW
    extra = ncols - per * NW
    base_c = wid * per + jnp.minimum(wid, extra)
    batches = [(h, off, min(QB, per - off))
               for h in range(D // SUB) for off in range(0, per, QB)]

    def fire_ins(b, slot):
        h, off, s = batches[b]
        ins = []
        for q in range(s):
            start = pl.multiple_of((base_c + off + q) * LANES, LANES)
            ins.append(pltpu.async_copy(
                src3.at[h, :, pl.ds(start, LANES)], buf.at[slot, q], sem_in))
        return ins

    outs = []
    ins_cur = fire_ins(0, 0)
    for b in range(len(batches)):
        slot = b % 2
        ins_next = None
        if b + 1 < len(batches):
            if b >= 1:
                outs[b - 1].wait()
            ins_next = fire_ins(b + 1, 1 - slot)
        for cp in ins_cur:
            cp.wait()
        h, off, s = batches[b]
        outs.append(pltpu.async_copy(
            buf.at[slot, pl.ds(0, s)],
            dst.at[pl.ds(h * ncols + base_c + off, s)], sem_out))
        ins_cur = ins_next
    for cp in outs[-2:]:
        cp.wait()

    @pl.when(wid < extra)
    def _():
        for h in range(D // SUB):
            c = base_c + per
            start = pl.multiple_of(c * LANES, LANES)
            pltpu.async_copy(
                src3.at[h, :, pl.ds(start, LANES)], buf.at[0, 0],
                sem_in).wait()
            pltpu.async_copy(
                buf.at[0, pl.ds(0, 1)],
                dst.at[pl.ds(h * ncols + c, 1)], sem_out).wait()


def _dump_body(tf_hbm, wf_hbm, tf_dump, wf_dump, buf, sem_in, sem_out):
    cid = lax.axis_index("c")
    sid = lax.axis_index("s")
    wid = sid * N_CORES + cid
    n_task = tf_hbm.shape[1]
    n_worker = wf_hbm.shape[1]
    tf3 = tf_hbm.reshape(D // SUB, SUB, n_task)
    wf3 = wf_hbm.reshape(D // SUB, SUB, n_worker)

    _dump_table(tf3, tf_dump, n_task, wid, buf, sem_in, sem_out)
    _dump_table(wf3, wf_dump, n_worker, wid, buf, sem_in, sem_out)


def _make_gather_body(n_task, n_worker):
    ct = [(k // SUB) * _ncols(n_task) * TILE_WORDS + (k % SUB) * LANES
          for k in range(D)]
    cw = [(k // SUB) * _ncols(n_worker) * TILE_WORDS + (k % SUB) * LANES
          for k in range(D)]

    def body(task_hbm, worker_hbm, tf_dump, wf_dump, out_hbm,
             raw_t, raw_w, idx_t, idx_w, dst_t, dst_w, out_v, sem_t, sem_w):
        wid = lax.axis_index("s") * N_CORES + lax.axis_index("c")
        base = wid * BPW
        pltpu.sync_copy(task_hbm.at[pl.ds(base, BPW)], raw_t)
        pltpu.sync_copy(worker_hbm.at[pl.ds(base, BPW)], raw_w)

        def expand(g, carry):
            rt = raw_t[pl.ds(g * D, D)]
            rw = raw_w[pl.ds(g * D, D)]
            ft = ((rt >> 7) << 10) + (rt & (LANES - 1))
            fw = ((rw >> 7) << 10) + (rw & (LANES - 1))
            for k in range(D):
                idx_t[pl.ds(k * BPW + g * D, D)] = ft + ct[k]
                idx_w[pl.ds(k * BPW + g * D, D)] = fw + cw[k]
            return carry

        lax.fori_loop(0, NGROUP, expand, 0)

        copies = []
        for k in range(D):
            for c in range(NCHUNK):
                off = k * BPW + c * CHUNK
                copies.append(pltpu.async_copy(
                    tf_dump.at[idx_t.at[pl.ds(off, CHUNK)]],
                    dst_t.at[pl.ds(off, CHUNK)], sem_t))
                copies.append(pltpu.async_copy(
                    wf_dump.at[idx_w.at[pl.ds(off, CHUNK)]],
                    dst_w.at[pl.ds(off, CHUNK)], sem_w))
        for cp in copies:
            cp.wait()

        def dot(g, carry):
            acc = jnp.zeros((D,), jnp.float32)
            for k in range(D):
                a = dst_t[pl.ds(k * BPW + g * D, D)]
                b = dst_w[pl.ds(k * BPW + g * D, D)]
                acc = acc + a * b
            out_v[pl.ds(g * D, D)] = 1.0 / (1.0 + jnp.exp(-acc))
            return carry

        lax.fori_loop(0, NGROUP, dot, 0)

        pltpu.sync_copy(out_v, out_hbm.at[pl.ds(base, BPW)])

    return body


@jax.jit
def kernel(task, worker, task_factors, worker_factors):
    n_task, _ = task_factors.shape
    n_worker, _ = worker_factors.shape
    nwords_t = (D // SUB) * _ncols(n_task) * TILE_WORDS
    nwords_w = (D // SUB) * _ncols(n_worker) * TILE_WORDS
    tf_t = task_factors.T
    wf_t = worker_factors.T

    mesh = plsc.VectorSubcoreMesh(core_axis_name="c", subcore_axis_name="s")

    dump = functools.partial(
        pl.kernel,
        out_type=(
            jax.ShapeDtypeStruct((nwords_t // TILE_WORDS, SUB, LANES), jnp.float32),
            jax.ShapeDtypeStruct((nwords_w // TILE_WORDS, SUB, LANES), jnp.float32),
        ),
        mesh=mesh,
        scratch_types=[
            pltpu.VMEM((2, QB, SUB, LANES), jnp.float32),
            pltpu.SemaphoreType.DMA,
            pltpu.SemaphoreType.DMA,
        ],
        compiler_params=pltpu.CompilerParams(needs_layout_passes=False),
    )(_dump_body)
    tf_dump, wf_dump = dump(tf_t, wf_t)
    tf_dump = tf_dump.reshape(nwords_t)
    wf_dump = wf_dump.reshape(nwords_w)

    gather = functools.partial(
        pl.kernel,
        out_type=jax.ShapeDtypeStruct((B,), jnp.float32),
        mesh=mesh,
        scratch_types=[
            pltpu.VMEM((BPW,), jnp.int32),
            pltpu.VMEM((BPW,), jnp.int32),
            pltpu.VMEM((D * BPW,), jnp.int32),
            pltpu.VMEM((D * BPW,), jnp.int32),
            pltpu.VMEM((D * BPW,), jnp.float32),
            pltpu.VMEM((D * BPW,), jnp.float32),
            pltpu.VMEM((BPW,), jnp.float32),
            pltpu.SemaphoreType.DMA,
            pltpu.SemaphoreType.DMA,
        ],
        compiler_params=pltpu.CompilerParams(
            needs_layout_passes=False, use_tc_tiling_on_sc=False),
    )(_make_gather_body(n_task, n_worker))
    return gather(task, worker, tf_dump, wf_dump)
